# Initial kernel scaffold; baseline (speedup 1.0000x reference)
import jax
import jax.numpy as jnp
from jax.experimental import pallas as pl


def kernel(queries, keys, temp, k):
    raise NotImplementedError("write your pallas kernel here")


# speedup vs baseline: 3.6710x; 3.6710x over previous
import functools

import jax
import jax.numpy as jnp
from jax.experimental import pallas as pl
from jax.experimental.pallas import tpu as pltpu
from jax.experimental.pallas import tpu_sc as plsc

Q = 1024
D = 1024
K = 100000
BS = 128
NB = 784
BN = 512
NBS = BN // BS
NSTEP = NB // NBS
KTOP = 50
KBLK = 64
NEG = float("-inf")
IMAX = 2**31 - 1


def _scores_body(q_ref, k_ref, t_ref, s_ref, bm_ref, qn_ref):
    j = pl.program_id(0)

    @pl.when(j == 0)
    def _():
        q = q_ref[...]
        qn_ref[...] = q / (jnp.sqrt(jnp.sum(q * q, axis=1, keepdims=True)) + 1e-8)

    kb = k_ref[...]
    kn = kb / (jnp.sqrt(jnp.sum(kb * kb, axis=1, keepdims=True)) + 1e-8)
    s = jax.lax.dot_general(qn_ref[...], kn, (((1,), (1,)), ((), ())),
                            preferred_element_type=jnp.float32)
    s = s / t_ref[...]
    gcol = j * BN + jax.lax.broadcasted_iota(jnp.int32, (Q, BN), 1)
    s = jnp.where(gcol < K, s, NEG)
    parts = []
    maxes = []
    for i in range(NBS):
        blk = s[:, i * BS:(i + 1) * BS]
        parts.append(blk[None])
        maxes.append(jnp.max(blk, axis=1, keepdims=True)[None])
    s_ref[...] = jnp.concatenate(parts, axis=0)
    bm_ref[...] = jnp.concatenate(maxes, axis=0)


def _scores_call(queries, keys, temp2d, interpret=False):
    return pl.pallas_call(
        _scores_body,
        grid=(NSTEP,),
        in_specs=[
            pl.BlockSpec((Q, D), lambda j: (0, 0)),
            pl.BlockSpec((BN, D), lambda j: (j, 0)),
            pl.BlockSpec((1, 1), lambda j: (0, 0)),
        ],
        out_specs=[
            pl.BlockSpec((NBS, Q, BS), lambda j: (j, 0, 0)),
            pl.BlockSpec((NBS, Q, 1), lambda j: (j, 0, 0)),
        ],
        out_shape=[
            jax.ShapeDtypeStruct((NB, Q, BS), jnp.float32),
            jax.ShapeDtypeStruct((NB, Q, 1), jnp.float32),
        ],
        scratch_shapes=[pltpu.VMEM((Q, D), jnp.float32)],
        interpret=interpret,
    )(queries, keys, temp2d)


def _select_body(bm_ref, f_ref, b_ref):
    col = jax.lax.broadcasted_iota(jnp.int32, (Q, NB), 1)
    sel = jax.lax.broadcasted_iota(jnp.int32, (Q, KBLK), 1)

    def step(t, carry):
        x, acc_b = carry
        m = jnp.max(x, axis=1, keepdims=True)
        ci = jnp.min(jnp.where(x == m, col, IMAX), axis=1, keepdims=True)
        acc_b = jnp.where(sel == t, ci, acc_b)
        x = jnp.where(col == ci, NEG, x)
        return x, acc_b

    _, acc_b = jax.lax.fori_loop(
        0, KBLK, step, (bm_ref[...], jnp.zeros((Q, KBLK), jnp.int32)))
    r = jax.lax.broadcasted_iota(jnp.int32, (Q, KBLK), 0)
    f_ref[...] = acc_b * Q + r
    b_ref[...] = acc_b


def _select_call(bm, interpret=False):
    return pl.pallas_call(
        _select_body,
        in_specs=[pl.BlockSpec((Q, NB), lambda: (0, 0))],
        out_specs=[
            pl.BlockSpec((Q, KBLK), lambda: (0, 0)),
            pl.BlockSpec((Q, KBLK), lambda: (0, 0)),
        ],
        out_shape=[
            jax.ShapeDtypeStruct((Q, KBLK), jnp.int32),
            jax.ShapeDtypeStruct((Q, KBLK), jnp.int32),
        ],
        interpret=interpret,
    )(bm)


_GROWS = Q * KBLK
_NWORK = 32
_RPW = _GROWS // _NWORK
_CH = 128
_NCH = _RPW // _CH


def _gather_body(tab_hbm, idx_hbm, out_hbm, idx_v, row_v, sem):
    w = jax.lax.axis_index("s") * 2 + jax.lax.axis_index("c")
    base = w * _RPW
    for c in range(_NCH):
        off = base + c * _CH
        pltpu.sync_copy(idx_hbm.at[pl.ds(off, _CH)], idx_v)
        pltpu.async_copy(tab_hbm.at[idx_v], row_v, sem).wait()
        pltpu.sync_copy(row_v, out_hbm.at[pl.ds(off, _CH)])


def _gather_call(table, fidx_flat):
    fn = functools.partial(
        pl.kernel,
        out_type=jax.ShapeDtypeStruct((_GROWS, BS), jnp.float32),
        mesh=plsc.VectorSubcoreMesh(core_axis_name="c", subcore_axis_name="s"),
        scratch_types=[
            pltpu.VMEM((_CH,), jnp.int32),
            pltpu.VMEM((_CH, BS), jnp.float32),
            pltpu.SemaphoreType.DMA,
        ],
    )(_gather_body)
    return fn(table, fidx_flat)


_RB = 32


def _final_body(c_ref, b_ref, p_ref, i_ref):
    g = b_ref[...] * BS + jax.lax.broadcasted_iota(jnp.int32, (_RB, KBLK, BS), 2)
    sel = jax.lax.broadcasted_iota(jnp.int32, (_RB, KBLK, 1), 1)

    def step(t, carry):
        v, acc_v, acc_i = carry
        m = jnp.max(jnp.max(v, axis=2, keepdims=True), axis=1, keepdims=True)
        cand = jnp.where(v == m, g, IMAX)
        ci = jnp.min(jnp.min(cand, axis=2, keepdims=True), axis=1, keepdims=True)
        acc_v = jnp.where(sel == t, m, acc_v)
        acc_i = jnp.where(sel == t, ci, acc_i)
        v = jnp.where(g == ci, NEG, v)
        return v, acc_v, acc_i

    _, acc_v, acc_i = jax.lax.fori_loop(
        0, KTOP, step,
        (c_ref[...].reshape(_RB, KBLK, BS),
         jnp.full((_RB, KBLK, 1), NEG, jnp.float32),
         jnp.zeros((_RB, KBLK, 1), jnp.int32)))
    e = jnp.exp(acc_v - acc_v[:, 0:1, :])
    p = e / jnp.sum(e, axis=1, keepdims=True)
    p_ref[...] = p[:, :KTOP, :]
    i_ref[...] = acc_i[:, :KTOP, :]


def _final_call(cand, bidx3, interpret=False):
    return pl.pallas_call(
        _final_body,
        grid=(Q // _RB,),
        in_specs=[
            pl.BlockSpec((_RB * KBLK, BS), lambda r: (r, 0)),
            pl.BlockSpec((_RB, KBLK, 1), lambda r: (r, 0, 0)),
        ],
        out_specs=[
            pl.BlockSpec((_RB, KTOP, 1), lambda r: (r, 0, 0)),
            pl.BlockSpec((_RB, KTOP, 1), lambda r: (r, 0, 0)),
        ],
        out_shape=[
            jax.ShapeDtypeStruct((Q, KTOP, 1), jnp.float32),
            jax.ShapeDtypeStruct((Q, KTOP, 1), jnp.int32),
        ],
        interpret=interpret,
    )(cand, bidx3)


def kernel(queries, keys, temp, k):
    del k
    temp2d = jnp.asarray(temp, jnp.float32).reshape(1, 1)
    scores3, bmax3 = _scores_call(queries, keys, temp2d)
    bm = bmax3.reshape(NB, Q).T
    fidx, bidx = _select_call(bm)
    table = scores3.reshape(NB * Q, BS)
    cand = _gather_call(table, fidx.reshape(_GROWS))
    probs3, idx3 = _final_call(cand, bidx.reshape(Q, KBLK, 1))
    return probs3.reshape(Q, KTOP), idx3.reshape(Q, KTOP)
